# baseline (speedup 1.0000x reference)
import functools

import jax
import jax.numpy as jnp
from jax import lax
from jax.experimental import pallas as pl
from jax.experimental.pallas import tpu as pltpu
from jax.experimental.pallas import tpu_sc as plsc

ZCH = 1000000
EMB = 64
N = 16384 * 20
NC = 2
NS = 16
LANES = 16
NW = NC * NS
NPW = N // NW
CHUNK = 128
NCHUNK = NPW // CHUNK

_mesh = plsc.VectorSubcoreMesh(core_axis_name="c", subcore_axis_name="s")


@functools.partial(
    pl.kernel,
    out_type=[
        jax.ShapeDtypeStruct((N,), jnp.int32),
        jax.ShapeDtypeStruct((N,), jnp.int32),
        jax.ShapeDtypeStruct((NW, LANES), jnp.float32),
    ],
    mesh=_mesh,
    compiler_params=pltpu.CompilerParams(use_tc_tiling_on_sc=False),
    scratch_types=[
        pltpu.VMEM((CHUNK,), jnp.int32),
        pltpu.VMEM((CHUNK, EMB), jnp.float32),
        pltpu.VMEM((LANES,), jnp.float32),
        pltpu.SemaphoreType.DMA,
    ],
)
def _sc_gather_reduce(v0_hbm, v1_hbm, t0_hbm, t1_hbm,
                      r0_hbm, r1_hbm, psum_hbm,
                      idx_v, rows_v, acc_v, sem):
    wid = lax.axis_index("s") * NC + lax.axis_index("c")
    base = wid * NPW

    def feature(v_hbm, t_hbm, r_hbm, a4):
        def chunk_body(c, a4):
            off = base + c * CHUNK
            pltpu.sync_copy(v_hbm.at[pl.ds(off, CHUNK)], idx_v)
            for k in range(CHUNK // LANES):
                sl = pl.ds(k * LANES, LANES)
                idx_v[sl] = lax.rem(idx_v[sl], jnp.full((LANES,), ZCH, jnp.int32))
            pltpu.sync_copy(idx_v, r_hbm.at[pl.ds(off, CHUNK)])
            pltpu.async_copy(t_hbm.at[idx_v], rows_v, sem).wait()
            def row(i, a4):
                a0, a1, a2, a3 = a4
                return (a0 + rows_v[i, pl.ds(0, LANES)],
                        a1 + rows_v[i, pl.ds(LANES, LANES)],
                        a2 + rows_v[i, pl.ds(2 * LANES, LANES)],
                        a3 + rows_v[i, pl.ds(3 * LANES, LANES)])
            return lax.fori_loop(0, CHUNK, row, a4)
        return lax.fori_loop(0, NCHUNK, chunk_body, a4)

    zero = jnp.zeros((LANES,), jnp.float32)
    a4 = (zero, zero, zero, zero)
    a4 = feature(v0_hbm, t0_hbm, r0_hbm, a4)
    a4 = feature(v1_hbm, t1_hbm, r1_hbm, a4)
    acc_v[...] = a4[0] + a4[1] + a4[2] + a4[3]
    pltpu.sync_copy(acc_v, psum_hbm.at[wid])


def kernel(values_0, values_1, table_0, table_1):
    remapped_0, remapped_1, psum = _sc_gather_reduce(
        values_0, values_1, table_0, table_1)
    loss = jnp.sum(psum) / jnp.float32(2 * N * EMB)
    return (loss, remapped_0, remapped_1)


# speedup vs baseline: 1.1776x; 1.1776x over previous
import functools

import jax
import jax.numpy as jnp
from jax import lax
from jax.experimental import pallas as pl
from jax.experimental.pallas import tpu as pltpu
from jax.experimental.pallas import tpu_sc as plsc

ZCH = 1000000
EMB = 64
N = 16384 * 20
NC = 2
NS = 16
LANES = 16
NW = NC * NS
NPW = N // NW
CHUNK = 128
NCHUNK = NPW // CHUNK
NBUF = 4

_mesh = plsc.VectorSubcoreMesh(core_axis_name="c", subcore_axis_name="s")


@functools.partial(
    pl.kernel,
    out_type=[
        jax.ShapeDtypeStruct((N // CHUNK, CHUNK), jnp.int32),
        jax.ShapeDtypeStruct((N // CHUNK, CHUNK), jnp.int32),
        jax.ShapeDtypeStruct((NW, LANES), jnp.float32),
    ],
    mesh=_mesh,
    compiler_params=pltpu.CompilerParams(use_tc_tiling_on_sc=False),
    scratch_types=[
        pltpu.VMEM((NCHUNK, CHUNK), jnp.int32),
        pltpu.VMEM((NCHUNK, CHUNK), jnp.int32),
        pltpu.VMEM((NBUF, CHUNK, EMB), jnp.float32),
        pltpu.VMEM((LANES,), jnp.float32),
        [pltpu.SemaphoreType.DMA] * NBUF,
        pltpu.SemaphoreType.DMA,
    ],
)
def _sc_gather_reduce(v0_hbm, v1_hbm, t0_hbm, t1_hbm,
                      r0_hbm, r1_hbm, psum_hbm,
                      idx0_v, idx1_v, rows_v, acc_v, gsem, wsem):
    wid = lax.axis_index("s") * NC + lax.axis_index("c")
    rbase = wid * NCHUNK

    pltpu.sync_copy(v0_hbm.at[pl.ds(rbase, NCHUNK)], idx0_v)
    pltpu.sync_copy(v1_hbm.at[pl.ds(rbase, NCHUNK)], idx1_v)

    zmod = jnp.full((LANES,), ZCH, jnp.int32)
    def rem_row(i, _):
        for k in range(CHUNK // LANES):
            sl = pl.ds(k * LANES, LANES)
            idx0_v[i, sl] = lax.rem(idx0_v[i, sl], zmod)
            idx1_v[i, sl] = lax.rem(idx1_v[i, sl], zmod)
        return 0
    lax.fori_loop(0, NCHUNK, rem_row, 0)
    w0 = pltpu.async_copy(idx0_v, r0_hbm.at[pl.ds(rbase, NCHUNK)], wsem)
    w1 = pltpu.async_copy(idx1_v, r1_hbm.at[pl.ds(rbase, NCHUNK)], wsem)

    def issue(t_hbm, idx_v, c, b, add):
        pltpu.async_copy(t_hbm.at[idx_v.at[c]], rows_v.at[b], gsem[b], add=add)

    def drain(b):
        pltpu.make_async_copy(t0_hbm.at[idx0_v.at[0]], rows_v.at[b], gsem[b]).wait()

    for b in range(NBUF):
        issue(t0_hbm, idx0_v, b, b, False)

    def f0_body(it, _):
        c0 = it * NBUF
        for b in range(NBUF):
            drain(b)
            issue(t0_hbm, idx0_v, c0 + b, b, True)
        return 0
    lax.fori_loop(1, NCHUNK // NBUF, f0_body, 0)

    def f1_body(it, _):
        c0 = it * NBUF
        for b in range(NBUF):
            drain(b)
            issue(t1_hbm, idx1_v, c0 + b, b, True)
        return 0
    lax.fori_loop(0, NCHUNK // NBUF, f1_body, 0)

    for b in range(NBUF):
        drain(b)

    def row(i, a4):
        a0, a1, a2, a3 = a4
        b = i // CHUNK
        r = i - b * CHUNK
        return (a0 + rows_v[b, r, pl.ds(0, LANES)],
                a1 + rows_v[b, r, pl.ds(LANES, LANES)],
                a2 + rows_v[b, r, pl.ds(2 * LANES, LANES)],
                a3 + rows_v[b, r, pl.ds(3 * LANES, LANES)])
    zero = jnp.zeros((LANES,), jnp.float32)
    a4 = lax.fori_loop(0, NBUF * CHUNK, row, (zero, zero, zero, zero))
    acc_v[...] = a4[0] + a4[1] + a4[2] + a4[3]
    pltpu.sync_copy(acc_v, psum_hbm.at[wid])
    w0.wait()
    w1.wait()


def kernel(values_0, values_1, table_0, table_1):
    v0 = values_0.reshape(N // CHUNK, CHUNK)
    v1 = values_1.reshape(N // CHUNK, CHUNK)
    remapped_0, remapped_1, psum = _sc_gather_reduce(v0, v1, table_0, table_1)
    loss = jnp.sum(psum) / jnp.float32(2 * N * EMB)
    return (loss, remapped_0.reshape(N), remapped_1.reshape(N))


# speedup vs baseline: 1.1792x; 1.0014x over previous
import functools

import jax
import jax.numpy as jnp
from jax import lax
from jax.experimental import pallas as pl
from jax.experimental.pallas import tpu as pltpu
from jax.experimental.pallas import tpu_sc as plsc

ZCH = 1000000
EMB = 64
N = 16384 * 20
NC = 2
NS = 16
LANES = 16
NW = NC * NS
NPW = N // NW
CHUNK = 128
NCHUNK = NPW // CHUNK
NBUF = 4

_mesh = plsc.VectorSubcoreMesh(core_axis_name="c", subcore_axis_name="s")


@functools.partial(
    pl.kernel,
    out_type=[
        jax.ShapeDtypeStruct((N,), jnp.int32),
        jax.ShapeDtypeStruct((N,), jnp.int32),
        jax.ShapeDtypeStruct((NW, LANES), jnp.float32),
    ],
    mesh=_mesh,
    compiler_params=pltpu.CompilerParams(use_tc_tiling_on_sc=False),
    scratch_types=[
        pltpu.VMEM((NPW,), jnp.int32),
        pltpu.VMEM((NPW,), jnp.int32),
        pltpu.VMEM((NBUF, CHUNK, EMB), jnp.float32),
        pltpu.VMEM((LANES,), jnp.float32),
        [pltpu.SemaphoreType.DMA] * NBUF,
        pltpu.SemaphoreType.DMA,
    ],
)
def _sc_gather_reduce(v0_hbm, v1_hbm, t0_hbm, t1_hbm,
                      r0_hbm, r1_hbm, psum_hbm,
                      idx0_v, idx1_v, rows_v, acc_v, gsem, wsem):
    wid = lax.axis_index("s") * NC + lax.axis_index("c")
    base = wid * NPW

    pltpu.sync_copy(v0_hbm.at[pl.ds(base, NPW)], idx0_v)
    pltpu.sync_copy(v1_hbm.at[pl.ds(base, NPW)], idx1_v)

    zmod = jnp.full((LANES,), ZCH, jnp.int32)
    def rem_blk(i, _):
        for k in range(8):
            sl = pl.ds(i * 8 * LANES + k * LANES, LANES)
            idx0_v[sl] = lax.rem(idx0_v[sl], zmod)
            idx1_v[sl] = lax.rem(idx1_v[sl], zmod)
        return 0
    lax.fori_loop(0, NPW // (8 * LANES), rem_blk, 0)
    w0 = pltpu.async_copy(idx0_v, r0_hbm.at[pl.ds(base, NPW)], wsem)
    w1 = pltpu.async_copy(idx1_v, r1_hbm.at[pl.ds(base, NPW)], wsem)

    def issue(t_hbm, idx_v, c, b, add):
        sl = pl.ds(c * CHUNK, CHUNK)
        pltpu.async_copy(t_hbm.at[idx_v.at[sl]], rows_v.at[b], gsem[b], add=add)

    def drain(b):
        pltpu.make_async_copy(
            t0_hbm.at[idx0_v.at[pl.ds(0, CHUNK)]], rows_v.at[b], gsem[b]).wait()

    for b in range(NBUF):
        issue(t0_hbm, idx0_v, b, b, False)

    def f0_body(it, _):
        for b in range(NBUF):
            drain(b)
            issue(t0_hbm, idx0_v, it * NBUF + b, b, True)
        return 0
    lax.fori_loop(1, NCHUNK // NBUF, f0_body, 0)

    def f1_body(it, _):
        for b in range(NBUF):
            drain(b)
            issue(t1_hbm, idx1_v, it * NBUF + b, b, True)
        return 0
    lax.fori_loop(0, NCHUNK // NBUF, f1_body, 0)

    for b in range(NBUF):
        drain(b)

    def row(i, a4):
        a0, a1, a2, a3 = a4
        b = i // CHUNK
        r = i - b * CHUNK
        return (a0 + rows_v[b, r, pl.ds(0, LANES)],
                a1 + rows_v[b, r, pl.ds(LANES, LANES)],
                a2 + rows_v[b, r, pl.ds(2 * LANES, LANES)],
                a3 + rows_v[b, r, pl.ds(3 * LANES, LANES)])
    zero = jnp.zeros((LANES,), jnp.float32)
    a4 = lax.fori_loop(0, NBUF * CHUNK, row, (zero, zero, zero, zero))
    acc_v[...] = a4[0] + a4[1] + a4[2] + a4[3]
    pltpu.sync_copy(acc_v, psum_hbm.at[wid])
    w0.wait()
    w1.wait()


def kernel(values_0, values_1, table_0, table_1):
    remapped_0, remapped_1, psum = _sc_gather_reduce(
        values_0, values_1, table_0, table_1)
    loss = jnp.sum(psum) / jnp.float32(2 * N * EMB)
    return (loss, remapped_0, remapped_1)


# speedup vs baseline: 6.9137x; 5.8629x over previous
import functools

import jax
import jax.numpy as jnp
from jax import lax
from jax.experimental import pallas as pl
from jax.experimental.pallas import tpu as pltpu
from jax.experimental.pallas import tpu_sc as plsc

ZCH = 1000000
EMB = 64
N = 16384 * 20
NS = 16
LANES = 16
NROW = N // 128
RPT = NROW // NS
HSL = 62504
HSL_LAST = ZCH - 15 * HSL
HPAD = 16 * HSL

_mesh = plsc.VectorSubcoreMesh(core_axis_name="c", subcore_axis_name="s")


@functools.partial(
    pl.kernel,
    out_type=[
        jax.ShapeDtypeStruct((NROW, 128), jnp.int32),
        jax.ShapeDtypeStruct((NROW, 128), jnp.int32),
        jax.ShapeDtypeStruct((ZCH,), jnp.float32),
        jax.ShapeDtypeStruct((ZCH,), jnp.float32),
    ],
    mesh=_mesh,
    compiler_params=pltpu.CompilerParams(use_tc_tiling_on_sc=False),
    scratch_types=[
        pltpu.VMEM((RPT, 128), jnp.int32),
        pltpu.VMEM((128,), jnp.float32),
        pltpu.VMEM_SHARED((HPAD,), jnp.float32),
        pltpu.SemaphoreType.DMA,
        pltpu.SemaphoreType.DMA,
        pltpu.SemaphoreType.DMA,
    ],
)
def _sc_hist(v0_hbm, v1_hbm, zeros_hbm, r0_hbm, r1_hbm, h0_hbm, h1_hbm,
             idx_v, ones_v, hist_s, zsem, wsem, ssem):
    cid = lax.axis_index("c")
    tid = lax.axis_index("s")
    hoff = tid * HSL

    for k in range(128 // LANES):
        ones_v[pl.ds(k * LANES, LANES)] = jnp.full((LANES,), 1.0, jnp.float32)

    pltpu.async_copy(zeros_hbm.at[pl.ds(0, HSL)],
                     hist_s.at[pl.ds(hoff, HSL)], zsem)

    def do_feature(v_hbm, r_hbm, h_hbm):
        base = tid * RPT
        pltpu.sync_copy(v_hbm.at[pl.ds(base, RPT)], idx_v)
        zmod = jnp.full((LANES,), ZCH, jnp.int32)

        def rem_row(i, _):
            for k in range(128 // LANES):
                sl = pl.ds(k * LANES, LANES)
                idx_v[i, sl] = lax.rem(idx_v[i, sl], zmod)
            return 0
        lax.fori_loop(0, RPT, rem_row, 0)
        w = pltpu.async_copy(idx_v, r_hbm.at[pl.ds(base, RPT)], wsem)

        pltpu.make_async_copy(zeros_hbm.at[pl.ds(0, HSL)],
                              hist_s.at[pl.ds(hoff, HSL)], zsem).wait()
        plsc.subcore_barrier()

        def blk(b, _):
            def fire(j, _):
                pltpu.async_copy(ones_v, hist_s.at[idx_v.at[b * 20 + j]],
                                 ssem, add=True)
                return 0
            lax.fori_loop(0, 20, fire, 0)

            def drain(j, _):
                pltpu.make_async_copy(ones_v, hist_s.at[idx_v.at[0]],
                                      ssem).wait()
                return 0
            lax.fori_loop(0, 20, drain, 0)
            return 0
        lax.fori_loop(0, RPT // 20, blk, 0)
        plsc.subcore_barrier()

        @pl.when(tid < NS - 1)
        def _():
            pltpu.sync_copy(hist_s.at[pl.ds(hoff, HSL)],
                            h_hbm.at[pl.ds(hoff, HSL)])
        @pl.when(tid == NS - 1)
        def _():
            pltpu.sync_copy(hist_s.at[pl.ds(hoff, HSL_LAST)],
                            h_hbm.at[pl.ds(hoff, HSL_LAST)])
        w.wait()

    @pl.when(cid == 0)
    def _():
        do_feature(v0_hbm, r0_hbm, h0_hbm)

    @pl.when(cid == 1)
    def _():
        do_feature(v1_hbm, r1_hbm, h1_hbm)


BLK = 16384
NBLK = (ZCH + BLK - 1) // BLK


def _rowsum_body(t0_ref, t1_ref, o0_ref, o1_ref):
    o0_ref[...] = jnp.sum(t0_ref[...], axis=0)
    o1_ref[...] = jnp.sum(t1_ref[...], axis=0)


_rowsum = pl.pallas_call(
    _rowsum_body,
    grid=(NBLK,),
    in_specs=[pl.BlockSpec((EMB, BLK), lambda i: (0, i)),
              pl.BlockSpec((EMB, BLK), lambda i: (0, i))],
    out_specs=[pl.BlockSpec((BLK,), lambda i: (i,)),
               pl.BlockSpec((BLK,), lambda i: (i,))],
    out_shape=[jax.ShapeDtypeStruct((ZCH,), jnp.float32)] * 2,
)


def _dot_body(h0_ref, r0_ref, h1_ref, r1_ref, acc_ref):
    i = pl.program_id(0)

    @pl.when(i == 0)
    def _():
        acc_ref[...] = jnp.zeros_like(acc_ref)

    prod = h0_ref[...] * r0_ref[...] + h1_ref[...] * r1_ref[...]
    p2 = prod.reshape(BLK // 128, 128)
    flat = (lax.broadcasted_iota(jnp.int32, (BLK // 128, 128), 0) * 128
            + lax.broadcasted_iota(jnp.int32, (BLK // 128, 128), 1))
    p2 = jnp.where(flat < ZCH - i * BLK, p2, 0.0)
    s = p2[0:8]
    for k in range(1, BLK // 1024):
        s = s + p2[k * 8:(k + 1) * 8]
    acc_ref[...] += s


_dot = pl.pallas_call(
    _dot_body,
    grid=(NBLK,),
    in_specs=[pl.BlockSpec((BLK,), lambda i: (i,))] * 4,
    out_specs=pl.BlockSpec((8, 128), lambda i: (0, 0)),
    out_shape=jax.ShapeDtypeStruct((8, 128), jnp.float32),
)


def kernel(values_0, values_1, table_0, table_1):
    v0 = values_0.reshape(NROW, 128)
    v1 = values_1.reshape(NROW, 128)
    zeros = jnp.zeros((HSL,), jnp.float32)
    r0, r1, h0, h1 = _sc_hist(v0, v1, zeros)
    rs0, rs1 = _rowsum(table_0.T, table_1.T)
    acc = _dot(h0, rs0, h1, rs1)
    loss = jnp.sum(acc) / jnp.float32(2 * N * EMB)
    return (loss, r0.reshape(N), r1.reshape(N))


# speedup vs baseline: 7.6846x; 1.1115x over previous
import functools

import jax
import jax.numpy as jnp
from jax import lax
from jax.experimental import pallas as pl
from jax.experimental.pallas import tpu as pltpu
from jax.experimental.pallas import tpu_sc as plsc

ZCH = 1000000
EMB = 64
N = 16384 * 20
NS = 16
LANES = 16
NROW = N // 128
RPT = NROW // NS
HSL = 62504
HSL_LAST = ZCH - 15 * HSL
HPAD = 16 * HSL

_mesh = plsc.VectorSubcoreMesh(core_axis_name="c", subcore_axis_name="s")


@functools.partial(
    pl.kernel,
    out_type=[
        jax.ShapeDtypeStruct((NROW, 128), jnp.int32),
        jax.ShapeDtypeStruct((NROW, 128), jnp.int32),
        jax.ShapeDtypeStruct((ZCH,), jnp.float32),
        jax.ShapeDtypeStruct((ZCH,), jnp.float32),
    ],
    mesh=_mesh,
    compiler_params=pltpu.CompilerParams(use_tc_tiling_on_sc=False),
    scratch_types=[
        pltpu.VMEM((RPT, 128), jnp.int32),
        pltpu.VMEM((128,), jnp.float32),
        pltpu.VMEM_SHARED((HPAD,), jnp.float32),
        pltpu.SemaphoreType.DMA,
        pltpu.SemaphoreType.DMA,
        pltpu.SemaphoreType.DMA,
    ],
)
def _sc_hist(v0_hbm, v1_hbm, zeros_hbm, r0_hbm, r1_hbm, h0_hbm, h1_hbm,
             idx_v, ones_v, hist_s, zsem, wsem, ssem):
    cid = lax.axis_index("c")
    tid = lax.axis_index("s")
    hoff = tid * HSL

    for k in range(128 // LANES):
        ones_v[pl.ds(k * LANES, LANES)] = jnp.full((LANES,), 1.0, jnp.float32)

    pltpu.async_copy(zeros_hbm.at[pl.ds(0, HSL)],
                     hist_s.at[pl.ds(hoff, HSL)], zsem)

    def do_feature(v_hbm, r_hbm, h_hbm):
        base = tid * RPT
        pltpu.sync_copy(v_hbm.at[pl.ds(base, RPT)], idx_v)
        zmod = jnp.full((LANES,), ZCH, jnp.int32)

        def rem_row(i, _):
            for k in range(128 // LANES):
                sl = pl.ds(k * LANES, LANES)
                idx_v[i, sl] = lax.rem(idx_v[i, sl], zmod)
            return 0
        lax.fori_loop(0, RPT, rem_row, 0)
        w = pltpu.async_copy(idx_v, r_hbm.at[pl.ds(base, RPT)], wsem)

        pltpu.make_async_copy(zeros_hbm.at[pl.ds(0, HSL)],
                              hist_s.at[pl.ds(hoff, HSL)], zsem).wait()
        plsc.subcore_barrier()

        def blk(b, _):
            def fire(j, _):
                pltpu.async_copy(ones_v, hist_s.at[idx_v.at[b * 20 + j]],
                                 ssem, add=True)
                return 0
            lax.fori_loop(0, 20, fire, 0)

            def drain(j, _):
                pltpu.make_async_copy(ones_v, hist_s.at[idx_v.at[0]],
                                      ssem).wait()
                return 0
            lax.fori_loop(0, 20, drain, 0)
            return 0
        lax.fori_loop(0, RPT // 20, blk, 0)
        plsc.subcore_barrier()

        @pl.when(tid < NS - 1)
        def _():
            pltpu.sync_copy(hist_s.at[pl.ds(hoff, HSL)],
                            h_hbm.at[pl.ds(hoff, HSL)])
        @pl.when(tid == NS - 1)
        def _():
            pltpu.sync_copy(hist_s.at[pl.ds(hoff, HSL_LAST)],
                            h_hbm.at[pl.ds(hoff, HSL_LAST)])
        w.wait()

    @pl.when(cid == 0)
    def _():
        do_feature(v0_hbm, r0_hbm, h0_hbm)

    @pl.when(cid == 1)
    def _():
        do_feature(v1_hbm, r1_hbm, h1_hbm)


BLK = 32768
NBLK = (ZCH + BLK - 1) // BLK


def _rowsum_body(t0_ref, t1_ref, o0_ref, o1_ref):
    o0_ref[...] = jnp.sum(t0_ref[...], axis=0)
    o1_ref[...] = jnp.sum(t1_ref[...], axis=0)


_rowsum = pl.pallas_call(
    _rowsum_body,
    grid=(NBLK,),
    in_specs=[pl.BlockSpec((EMB, BLK), lambda i: (0, i)),
              pl.BlockSpec((EMB, BLK), lambda i: (0, i))],
    out_specs=[pl.BlockSpec((BLK,), lambda i: (i,)),
               pl.BlockSpec((BLK,), lambda i: (i,))],
    out_shape=[jax.ShapeDtypeStruct((ZCH,), jnp.float32)] * 2,
)

DBLK = 65536
DNBLK = (ZCH + DBLK - 1) // DBLK


def _dot_body(h0_ref, r0_ref, h1_ref, r1_ref, acc_ref):
    i = pl.program_id(0)

    @pl.when(i == 0)
    def _():
        acc_ref[...] = jnp.zeros_like(acc_ref)

    prod = h0_ref[...] * r0_ref[...] + h1_ref[...] * r1_ref[...]
    p2 = prod.reshape(DBLK // 128, 128)

    def tree_sum(x):
        s = x[0:8]
        for k in range(1, DBLK // 1024):
            s = s + x[k * 8:(k + 1) * 8]
        return s

    @pl.when(i < DNBLK - 1)
    def _():
        acc_ref[...] += tree_sum(p2)

    @pl.when(i == DNBLK - 1)
    def _():
        flat = (lax.broadcasted_iota(jnp.int32, (DBLK // 128, 128), 0) * 128
                + lax.broadcasted_iota(jnp.int32, (DBLK // 128, 128), 1))
        acc_ref[...] += tree_sum(jnp.where(flat < ZCH - i * DBLK, p2, 0.0))


_dot = pl.pallas_call(
    _dot_body,
    grid=(DNBLK,),
    in_specs=[pl.BlockSpec((DBLK,), lambda i: (i,))] * 4,
    out_specs=pl.BlockSpec((8, 128), lambda i: (0, 0)),
    out_shape=jax.ShapeDtypeStruct((8, 128), jnp.float32),
)


def kernel(values_0, values_1, table_0, table_1):
    v0 = values_0.reshape(NROW, 128)
    v1 = values_1.reshape(NROW, 128)
    zeros = jnp.zeros((HSL,), jnp.float32)
    r0, r1, h0, h1 = _sc_hist(v0, v1, zeros)
    rs0, rs1 = _rowsum(table_0.T, table_1.T)
    acc = _dot(h0, rs0, h1, rs1)
    loss = jnp.sum(acc) / jnp.float32(2 * N * EMB)
    return (loss, r0.reshape(N), r1.reshape(N))
